# baseline (speedup 1.0000x reference)
import jax
import jax.numpy as jnp
from jax import lax
from jax.experimental import pallas as pl
from jax.experimental.pallas import tpu as pltpu
from jax.experimental.pallas import tpu_sc as plsc

BATCH = 16384
EMB2 = 64
NC, NS, LANES = 2, 16, 16
NW = NC * NS
BPW = BATCH // NW
CHUNK = 128
NCH = BPW // CHUNK
ENT_VOCAB = 1000000

EBLK = 65536
NEB = 14
TC_ROWS = NEB * (EBLK // 128)
SCN = 2 * EBLK
SC_CPW = SCN // NW
SC_ROWS = SCN // 128
SCOL = 512


def _ent_score_body(tnat_ref, w_ref, relp3_ref, b_ref, o_ref, rsc_ref):
    w_obj = w_ref[EMB2:2 * EMB2, 0]
    x = tnat_ref[...].reshape(EMB2, EBLK // 128, 128)
    o_ref[...] = lax.dot_general(
        w_obj, x, dimension_numbers=(((0,), (0,)), ((), ())),
        preferred_element_type=jnp.float32)

    @pl.when(pl.program_id(0) == 0)
    def _():
        w_rel = w_ref[0:EMB2, 0]
        rsc_ref[...] = (
            lax.dot_general(relp3_ref[...], w_rel,
                            dimension_numbers=(((2,), (0,)), ((), ())),
                            preferred_element_type=jnp.float32)
            + b_ref[0, 0]
        )


@jax.jit
def _tc_scores(tnat, W, relp3, b2):
    return pl.pallas_call(
        _ent_score_body,
        grid=(NEB,),
        in_specs=[
            pl.BlockSpec((EMB2, EBLK), lambda i: (0, i + 2)),
            pl.BlockSpec((2 * EMB2, 1), lambda i: (0, 0)),
            pl.BlockSpec((8, 128, EMB2), lambda i: (0, 0, 0)),
            pl.BlockSpec((1, 1), lambda i: (0, 0)),
        ],
        out_specs=[
            pl.BlockSpec((EBLK // 128, 128), lambda i: (i, 0)),
            pl.BlockSpec((8, 128), lambda i: (0, 0)),
        ],
        out_shape=[
            jax.ShapeDtypeStruct((TC_ROWS, 128), jnp.float32),
            jax.ShapeDtypeStruct((8, 128), jnp.float32),
        ],
        compiler_params=pltpu.CompilerParams(
            dimension_semantics=("parallel",)),
    )(tnat, W, relp3, b2)


def _scan_body(tnat_hbm, wb_hbm, esc_hbm, buf_v, out_v, wb_v):
    wid = lax.axis_index("s") * NC + lax.axis_index("c")
    col0 = wid * SC_CPW

    pltpu.sync_copy(wb_hbm, wb_v)
    w_vecs = [wb_v[pl.ds(k * LANES, LANES)] for k in range(8)]
    w_obj = [w_vecs[4 + d // LANES][d % LANES] for d in range(EMB2)]

    def chunk(c, carry):
        pltpu.sync_copy(tnat_hbm.at[:, pl.ds(col0 + c * SCOL, SCOL)], buf_v)

        def group(g, carry2):
            acc = jnp.zeros((LANES,), jnp.float32)
            for d in range(EMB2):
                acc = acc + buf_v[d, pl.ds(g * LANES, LANES)] * w_obj[d]
            out_v[pl.ds(c * SCOL + g * LANES, LANES)] = acc
            return carry2

        lax.fori_loop(0, SCOL // LANES, group, 0)
        return carry

    lax.fori_loop(0, SC_CPW // SCOL, chunk, 0)
    pltpu.sync_copy(out_v, esc_hbm.at[pl.ds(wid * SC_CPW, SC_CPW)])


@jax.jit
def _sc_scores(tnat, wb):
    mesh = plsc.VectorSubcoreMesh(core_axis_name="c", subcore_axis_name="s")
    run = pl.kernel(
        _scan_body,
        out_type=jax.ShapeDtypeStruct((SCN,), jnp.float32),
        mesh=mesh,
        compiler_params=pltpu.CompilerParams(needs_layout_passes=False),
        scratch_types=[
            pltpu.VMEM((EMB2, SCOL), jnp.float32),
            pltpu.VMEM((SC_CPW,), jnp.float32),
            pltpu.VMEM((8 * LANES,), jnp.float32),
        ],
    )
    return run(tnat, wb)


def _combine_body(qrhi_hbm, qrlo_hbm, qohi_hbm, qolo_hbm, rsc_hbm, esc_hbm,
                  out_hbm,
                  qrhi_v, qrlo_v, qohi_v, qolo_v, rsc_v, erows_v, out_v, sem):
    wid = lax.axis_index("s") * NC + lax.axis_index("c")
    base = wid * BPW

    pltpu.sync_copy(qrhi_hbm.at[wid], qrhi_v)
    pltpu.sync_copy(qrlo_hbm.at[wid], qrlo_v)
    pltpu.sync_copy(qohi_hbm.at[wid], qohi_v)
    pltpu.sync_copy(qolo_hbm.at[wid], qolo_v)
    pltpu.sync_copy(rsc_hbm, rsc_v)

    copies = [
        pltpu.async_copy(esc_hbm.at[qohi_v.at[c]],
                         erows_v.at[pl.ds(c * CHUNK, CHUNK)], sem)
        for c in range(NCH)
    ]
    for c in copies:
        c.wait()

    def group(g, carry):
        sl = pl.ds(g * LANES, LANES)
        e_vec = lax.iota(jnp.int32, LANES) + g * LANES
        acc = plsc.load_gather(rsc_v, [qrhi_v[sl], qrlo_v[sl]])
        acc = acc + plsc.load_gather(erows_v, [e_vec, qolo_v[sl]])
        out_v[sl] = acc
        return carry

    lax.fori_loop(0, BPW // LANES, group, 0)
    pltpu.sync_copy(out_v, out_hbm.at[pl.ds(base, BPW)])


@jax.jit
def _combine(qr_hi, qr_lo, qo_hi, qo_lo, rel_scores, ent_scores):
    mesh = plsc.VectorSubcoreMesh(core_axis_name="c", subcore_axis_name="s")
    run = pl.kernel(
        _combine_body,
        out_type=jax.ShapeDtypeStruct((BATCH,), jnp.float32),
        mesh=mesh,
        compiler_params=pltpu.CompilerParams(needs_layout_passes=False),
        scratch_types=[
            pltpu.VMEM((BPW,), jnp.int32),
            pltpu.VMEM((BPW,), jnp.int32),
            pltpu.VMEM((NCH, CHUNK), jnp.int32),
            pltpu.VMEM((BPW,), jnp.int32),
            pltpu.VMEM((8, 128), jnp.float32),
            pltpu.VMEM((BPW, 128), jnp.float32),
            pltpu.VMEM((BPW,), jnp.float32),
            pltpu.SemaphoreType.DMA,
        ],
    )
    return run(qr_hi, qr_lo, qo_hi, qo_lo, rel_scores, ent_scores)


def kernel(query_relation, query_object, relation_table, entity_table, W, b):
    qr = query_relation.astype(jnp.int32)
    qo = query_object.astype(jnp.int32)
    tnat = entity_table.T
    wb = W.reshape(4 * 32)
    relp3 = jnp.pad(relation_table, ((0, 24), (0, 0))).reshape(8, 128, EMB2)

    esc_sc = _sc_scores(tnat, wb)
    esc_tc, rel_scores = _tc_scores(tnat, W, relp3, b.reshape(1, 1))
    esc = jnp.concatenate(
        [esc_sc.reshape(SC_ROWS, 128), esc_tc], axis=0)

    row = qo >> 7
    out = _combine(
        (qr >> 7).reshape(NW, BPW), (qr & 127).reshape(NW, BPW),
        row.reshape(NW, NCH, CHUNK), (qo & 127).reshape(NW, BPW),
        rel_scores, esc)
    return out.reshape(BATCH, 1)


# speedup vs baseline: 1.0092x; 1.0092x over previous
import jax
import jax.numpy as jnp
from jax import lax
from jax.experimental import pallas as pl
from jax.experimental.pallas import tpu as pltpu
from jax.experimental.pallas import tpu_sc as plsc

BATCH = 16384
EMB2 = 64
NC, NS, LANES = 2, 16, 16
NW = NC * NS
BPW = BATCH // NW
CHUNK = 128
NCH = BPW // CHUNK
ENT_VOCAB = 1000000

EBLK = 65536
NEB = 12
TC_ROWS = NEB * (EBLK // 128)
SCN = 4 * EBLK
SC_CPW = SCN // NW
SC_ROWS = SCN // 128
SCOL = 512


def _ent_score_body(tnat_ref, w_ref, relp3_ref, b_ref, o_ref, rsc_ref):
    w_obj = w_ref[EMB2:2 * EMB2, 0]
    x = tnat_ref[...].reshape(EMB2, EBLK // 128, 128)
    o_ref[...] = lax.dot_general(
        w_obj, x, dimension_numbers=(((0,), (0,)), ((), ())),
        preferred_element_type=jnp.float32)

    @pl.when(pl.program_id(0) == 0)
    def _():
        w_rel = w_ref[0:EMB2, 0]
        rsc_ref[...] = (
            lax.dot_general(relp3_ref[...], w_rel,
                            dimension_numbers=(((2,), (0,)), ((), ())),
                            preferred_element_type=jnp.float32)
            + b_ref[0, 0]
        )


@jax.jit
def _tc_scores(tnat, W, relp3, b2):
    return pl.pallas_call(
        _ent_score_body,
        grid=(NEB,),
        in_specs=[
            pl.BlockSpec((EMB2, EBLK), lambda i: (0, i + 4)),
            pl.BlockSpec((2 * EMB2, 1), lambda i: (0, 0)),
            pl.BlockSpec((8, 128, EMB2), lambda i: (0, 0, 0)),
            pl.BlockSpec((1, 1), lambda i: (0, 0)),
        ],
        out_specs=[
            pl.BlockSpec((EBLK // 128, 128), lambda i: (i, 0)),
            pl.BlockSpec((8, 128), lambda i: (0, 0)),
        ],
        out_shape=[
            jax.ShapeDtypeStruct((TC_ROWS, 128), jnp.float32),
            jax.ShapeDtypeStruct((8, 128), jnp.float32),
        ],
        compiler_params=pltpu.CompilerParams(
            dimension_semantics=("parallel",)),
    )(tnat, W, relp3, b2)


def _scan_body(tnat_hbm, wb_hbm, esc_hbm, buf0_v, buf1_v, out_v, wb_v,
               sem0, sem1):
    wid = lax.axis_index("s") * NC + lax.axis_index("c")
    col0 = wid * SC_CPW

    pltpu.sync_copy(wb_hbm, wb_v)
    w_vecs = [wb_v[pl.ds(k * LANES, LANES)] for k in range(8)]
    w_obj = [w_vecs[4 + d // LANES][d % LANES] for d in range(EMB2)]

    nch = SC_CPW // SCOL
    bufs = (buf0_v, buf1_v)
    sems = (sem0, sem1)

    def start(c):
        return pltpu.async_copy(
            tnat_hbm.at[:, pl.ds(col0 + c * SCOL, SCOL)],
            bufs[c % 2], sems[c % 2])

    def compute(c, cp):
        cp.wait()
        buf = bufs[c % 2]

        def group(g, carry):
            acc = jnp.zeros((LANES,), jnp.float32)
            for d in range(EMB2):
                acc = acc + buf[d, pl.ds(g * LANES, LANES)] * w_obj[d]
            out_v[pl.ds(c * SCOL + g * LANES, LANES)] = acc
            return carry

        lax.fori_loop(0, SCOL // LANES, group, 0)

    cp = start(0)
    for c in range(nch):
        nxt = start(c + 1) if c + 1 < nch else None
        compute(c, cp)
        cp = nxt
    pltpu.sync_copy(out_v, esc_hbm.at[pl.ds(wid * SC_CPW, SC_CPW)])


@jax.jit
def _sc_scores(tnat, wb):
    mesh = plsc.VectorSubcoreMesh(core_axis_name="c", subcore_axis_name="s")
    run = pl.kernel(
        _scan_body,
        out_type=jax.ShapeDtypeStruct((SCN,), jnp.float32),
        mesh=mesh,
        compiler_params=pltpu.CompilerParams(needs_layout_passes=False),
        scratch_types=[
            pltpu.VMEM((EMB2, SCOL), jnp.float32),
            pltpu.VMEM((EMB2, SCOL), jnp.float32),
            pltpu.VMEM((SC_CPW,), jnp.float32),
            pltpu.VMEM((8 * LANES,), jnp.float32),
            pltpu.SemaphoreType.DMA,
            pltpu.SemaphoreType.DMA,
        ],
    )
    return run(tnat, wb)


def _combine_body(qrhi_hbm, qrlo_hbm, qohi_hbm, qolo_hbm, rsc_hbm, esc_hbm,
                  out_hbm,
                  qrhi_v, qrlo_v, qohi_v, qolo_v, rsc_v, erows_v, out_v, sem):
    wid = lax.axis_index("s") * NC + lax.axis_index("c")
    base = wid * BPW

    pltpu.sync_copy(qrhi_hbm.at[wid], qrhi_v)
    pltpu.sync_copy(qrlo_hbm.at[wid], qrlo_v)
    pltpu.sync_copy(qohi_hbm.at[wid], qohi_v)
    pltpu.sync_copy(qolo_hbm.at[wid], qolo_v)
    pltpu.sync_copy(rsc_hbm, rsc_v)

    copies = [
        pltpu.async_copy(esc_hbm.at[qohi_v.at[c]],
                         erows_v.at[pl.ds(c * CHUNK, CHUNK)], sem)
        for c in range(NCH)
    ]
    for c in copies:
        c.wait()

    def group(g, carry):
        sl = pl.ds(g * LANES, LANES)
        e_vec = lax.iota(jnp.int32, LANES) + g * LANES
        acc = plsc.load_gather(rsc_v, [qrhi_v[sl], qrlo_v[sl]])
        acc = acc + plsc.load_gather(erows_v, [e_vec, qolo_v[sl]])
        out_v[sl] = acc
        return carry

    lax.fori_loop(0, BPW // LANES, group, 0)
    pltpu.sync_copy(out_v, out_hbm.at[pl.ds(base, BPW)])


@jax.jit
def _combine(qr_hi, qr_lo, qo_hi, qo_lo, rel_scores, ent_scores):
    mesh = plsc.VectorSubcoreMesh(core_axis_name="c", subcore_axis_name="s")
    run = pl.kernel(
        _combine_body,
        out_type=jax.ShapeDtypeStruct((BATCH,), jnp.float32),
        mesh=mesh,
        compiler_params=pltpu.CompilerParams(needs_layout_passes=False),
        scratch_types=[
            pltpu.VMEM((BPW,), jnp.int32),
            pltpu.VMEM((BPW,), jnp.int32),
            pltpu.VMEM((NCH, CHUNK), jnp.int32),
            pltpu.VMEM((BPW,), jnp.int32),
            pltpu.VMEM((8, 128), jnp.float32),
            pltpu.VMEM((BPW, 128), jnp.float32),
            pltpu.VMEM((BPW,), jnp.float32),
            pltpu.SemaphoreType.DMA,
        ],
    )
    return run(qr_hi, qr_lo, qo_hi, qo_lo, rel_scores, ent_scores)


def kernel(query_relation, query_object, relation_table, entity_table, W, b):
    qr = query_relation.astype(jnp.int32)
    qo = query_object.astype(jnp.int32)
    tnat = entity_table.T
    wb = W.reshape(4 * 32)
    relp3 = jnp.pad(relation_table, ((0, 24), (0, 0))).reshape(8, 128, EMB2)

    esc_sc = _sc_scores(tnat, wb)
    esc_tc, rel_scores = _tc_scores(tnat, W, relp3, b.reshape(1, 1))
    esc = jnp.concatenate(
        [esc_sc.reshape(SC_ROWS, 128), esc_tc], axis=0)

    row = qo >> 7
    out = _combine(
        (qr >> 7).reshape(NW, BPW), (qr & 127).reshape(NW, BPW),
        row.reshape(NW, NCH, CHUNK), (qo & 127).reshape(NW, BPW),
        rel_scores, esc)
    return out.reshape(BATCH, 1)


# speedup vs baseline: 1.0259x; 1.0165x over previous
import jax
import jax.numpy as jnp
from jax import lax
from jax.experimental import pallas as pl
from jax.experimental.pallas import tpu as pltpu
from jax.experimental.pallas import tpu_sc as plsc

BATCH = 16384
EMB2 = 64
NC, NS, LANES = 2, 16, 16
NW = NC * NS
BPW = BATCH // NW
CHUNK = 128
NCH = BPW // CHUNK
ENT_VOCAB = 1000000

EBLK = 65536
NEB = 12
TC_ROWS = NEB * (EBLK // 128)
SCN = 4 * EBLK
SC_CPW = SCN // NW
SC_ROWS = SCN // 128
SCOL = 512


def _ent_score_body(tnat_ref, w_ref, relp3_ref, b_ref, o_ref, rsc_ref):
    w_obj = w_ref[EMB2:2 * EMB2, 0]
    x = tnat_ref[...].reshape(EMB2, EBLK // 128, 128)
    o_ref[...] = lax.dot_general(
        w_obj, x, dimension_numbers=(((0,), (0,)), ((), ())),
        preferred_element_type=jnp.float32)

    @pl.when(pl.program_id(0) == 0)
    def _():
        w_rel = w_ref[0:EMB2, 0]
        rsc_ref[...] = (
            lax.dot_general(relp3_ref[...], w_rel,
                            dimension_numbers=(((2,), (0,)), ((), ())),
                            preferred_element_type=jnp.float32)
            + b_ref[0, 0]
        )


@jax.jit
def _tc_scores(tnat, W, relp3, b2):
    return pl.pallas_call(
        _ent_score_body,
        grid=(NEB,),
        in_specs=[
            pl.BlockSpec((EMB2, EBLK), lambda i: (0, i + 4)),
            pl.BlockSpec((2 * EMB2, 1), lambda i: (0, 0)),
            pl.BlockSpec((8, 128, EMB2), lambda i: (0, 0, 0)),
            pl.BlockSpec((1, 1), lambda i: (0, 0)),
        ],
        out_specs=[
            pl.BlockSpec((EBLK // 128, 128), lambda i: (i, 0)),
            pl.BlockSpec((8, 128), lambda i: (0, 0)),
        ],
        out_shape=[
            jax.ShapeDtypeStruct((TC_ROWS, 128), jnp.float32),
            jax.ShapeDtypeStruct((8, 128), jnp.float32),
        ],
        compiler_params=pltpu.CompilerParams(
            dimension_semantics=("parallel",)),
    )(tnat, W, relp3, b2)


def _scan_body(tnat_hbm, wb_hbm, esc_hbm, buf0_v, buf1_v, out_v, wb_v,
               sem0, sem1):
    wid = lax.axis_index("s") * NC + lax.axis_index("c")
    col0 = wid * SC_CPW

    pltpu.sync_copy(wb_hbm, wb_v)
    w_vecs = [wb_v[pl.ds(k * LANES, LANES)] for k in range(8)]
    w_obj = [w_vecs[4 + d // LANES][d % LANES] for d in range(EMB2)]

    nch = SC_CPW // SCOL
    bufs = (buf0_v, buf1_v)
    sems = (sem0, sem1)

    def start(c):
        return pltpu.async_copy(
            tnat_hbm.at[:, pl.ds(col0 + c * SCOL, SCOL)],
            bufs[c % 2], sems[c % 2])

    def compute(c, cp):
        cp.wait()
        buf = bufs[c % 2]

        def group(g, carry):
            acc = jnp.zeros((LANES,), jnp.float32)
            for d in range(EMB2):
                acc = acc + buf[d, pl.ds(g * LANES, LANES)] * w_obj[d]
            out_v[pl.ds(c * SCOL + g * LANES, LANES)] = acc
            return carry

        lax.fori_loop(0, SCOL // LANES, group, 0)

    cp = start(0)
    for c in range(nch):
        nxt = start(c + 1) if c + 1 < nch else None
        compute(c, cp)
        cp = nxt
    pltpu.sync_copy(out_v, esc_hbm.at[pl.ds(wid * SC_CPW, SC_CPW)])


@jax.jit
def _sc_scores(tnat, wb):
    mesh = plsc.VectorSubcoreMesh(core_axis_name="c", subcore_axis_name="s")
    run = pl.kernel(
        _scan_body,
        out_type=jax.ShapeDtypeStruct((SCN,), jnp.float32),
        mesh=mesh,
        compiler_params=pltpu.CompilerParams(needs_layout_passes=False),
        scratch_types=[
            pltpu.VMEM((EMB2, SCOL), jnp.float32),
            pltpu.VMEM((EMB2, SCOL), jnp.float32),
            pltpu.VMEM((SC_CPW,), jnp.float32),
            pltpu.VMEM((8 * LANES,), jnp.float32),
            pltpu.SemaphoreType.DMA,
            pltpu.SemaphoreType.DMA,
        ],
    )
    return run(tnat, wb)


def _combine_body(qr_hbm, qo_hbm, rsc_hbm, esc_hbm, out_hbm,
                  qr_v, qo_v, qohi_v, rsc_v, erows_v, out_v, sem):
    wid = lax.axis_index("s") * NC + lax.axis_index("c")
    base = wid * BPW

    pltpu.sync_copy(qr_hbm.at[wid], qr_v)
    pltpu.sync_copy(qo_hbm.at[wid], qo_v)
    pltpu.sync_copy(rsc_hbm, rsc_v)

    for c in range(NCH):
        for g in range(CHUNK // LANES):
            sl = pl.ds(c * CHUNK + g * LANES, LANES)
            qohi_v[c, pl.ds(g * LANES, LANES)] = qo_v[sl] >> 7

    copies = [
        pltpu.async_copy(esc_hbm.at[qohi_v.at[c]],
                         erows_v.at[pl.ds(c * CHUNK, CHUNK)], sem)
        for c in range(NCH)
    ]
    for c in copies:
        c.wait()

    def group(g, carry):
        sl = pl.ds(g * LANES, LANES)
        e_vec = lax.iota(jnp.int32, LANES) + g * LANES
        qr16 = qr_v[sl]
        acc = plsc.load_gather(rsc_v, [qr16 >> 7, qr16 & 127])
        acc = acc + plsc.load_gather(erows_v, [e_vec, qo_v[sl] & 127])
        out_v[sl] = acc
        return carry

    lax.fori_loop(0, BPW // LANES, group, 0)
    pltpu.sync_copy(out_v, out_hbm.at[pl.ds(base, BPW)])


@jax.jit
def _combine(qr2, qo2, rel_scores, ent_scores):
    mesh = plsc.VectorSubcoreMesh(core_axis_name="c", subcore_axis_name="s")
    run = pl.kernel(
        _combine_body,
        out_type=jax.ShapeDtypeStruct((BATCH,), jnp.float32),
        mesh=mesh,
        compiler_params=pltpu.CompilerParams(needs_layout_passes=False),
        scratch_types=[
            pltpu.VMEM((BPW,), jnp.int32),
            pltpu.VMEM((BPW,), jnp.int32),
            pltpu.VMEM((NCH, CHUNK), jnp.int32),
            pltpu.VMEM((8, 128), jnp.float32),
            pltpu.VMEM((BPW, 128), jnp.float32),
            pltpu.VMEM((BPW,), jnp.float32),
            pltpu.SemaphoreType.DMA,
        ],
    )
    return run(qr2, qo2, rel_scores, ent_scores)


def kernel(query_relation, query_object, relation_table, entity_table, W, b):
    qr = query_relation.astype(jnp.int32)
    qo = query_object.astype(jnp.int32)
    tnat = entity_table.T
    wb = W.reshape(4 * 32)
    relp3 = jnp.pad(relation_table, ((0, 24), (0, 0))).reshape(8, 128, EMB2)

    esc_sc = _sc_scores(tnat, wb)
    esc_tc, rel_scores = _tc_scores(tnat, W, relp3, b.reshape(1, 1))
    esc = jnp.concatenate(
        [esc_sc.reshape(SC_ROWS, 128), esc_tc], axis=0)

    out = _combine(qr.reshape(NW, BPW), qo.reshape(NW, BPW), rel_scores, esc)
    return out.reshape(BATCH, 1)
